# baseline (speedup 1.0000x reference)
import functools

import jax
import jax.numpy as jnp
from jax.experimental import pallas as pl
from jax.experimental.pallas import tpu as pltpu

_GATHER_W = 8


def _gather_kernel(lidx_ref, *refs):
    del lidx_ref
    xrows, out_ref = refs[:_GATHER_W], refs[_GATHER_W]
    for j in range(_GATHER_W):
        out_ref[j, :] = xrows[j][0, 0, :]


def _batch_kernel(uidx_ref, lemb_ref, out_ref, *, n_user):
    tl = lemb_ref.shape[0]
    rows = jax.lax.broadcasted_iota(jnp.int32, (n_user, tl), 0)
    oh = (rows == uidx_ref[0]).astype(jnp.float32)
    sums = jnp.dot(oh, lemb_ref[...], preferred_element_type=jnp.float32)
    counts = jnp.sum(oh, axis=1, keepdims=True)
    has = counts > 0.0
    avg = sums / jnp.maximum(counts, 1.0)
    n_edge = jnp.maximum(jnp.sum(has.astype(jnp.float32)), 1.0)
    mean_b = jnp.sum(avg, axis=0, keepdims=True) / n_edge
    out_ref[0] = jnp.where(has, avg, mean_b)


def kernel(x_location, x_mobility_batch, x_text_batch, sorted_user, sorted_location):
    x_m_t = jnp.concatenate([x_mobility_batch, x_text_batch], axis=2)
    links0 = x_m_t[:, 0]
    batch, L, _ = links0.shape
    n_loc, D = x_location.shape
    n_user = sorted_user.shape[0]

    uidx = jnp.take(sorted_user, links0[..., 0]).astype(jnp.int32)
    lidx = jnp.take(sorted_location, links0[..., 1]).astype(jnp.int32)
    lidx_flat = lidx.reshape(batch * L)

    xloc3 = x_location.reshape(n_loc, 1, D)

    LB = batch * L
    n_steps = LB // _GATHER_W
    in_specs = [
        pl.BlockSpec((1, 1, D),
                     (lambda s, lidx_ref, j=j: (lidx_ref[_GATHER_W * s + j], 0, 0)))
        for j in range(_GATHER_W)
    ]
    link_emb = pl.pallas_call(
        _gather_kernel,
        out_shape=jax.ShapeDtypeStruct((LB, D), jnp.float32),
        grid_spec=pltpu.PrefetchScalarGridSpec(
            num_scalar_prefetch=1,
            grid=(n_steps,),
            in_specs=in_specs,
            out_specs=pl.BlockSpec((_GATHER_W, D), lambda s, lidx_ref: (s, 0)),
        ),
        compiler_params=pltpu.CompilerParams(
            dimension_semantics=("parallel",),
            vmem_limit_bytes=32 * 1024 * 1024),
    )(lidx_flat, *([xloc3] * _GATHER_W))

    body = functools.partial(_batch_kernel, n_user=n_user)
    out3 = pl.pallas_call(
        body,
        out_shape=jax.ShapeDtypeStruct((batch, n_user, D), jnp.float32),
        grid_spec=pltpu.PrefetchScalarGridSpec(
            num_scalar_prefetch=0,
            grid=(batch,),
            in_specs=[pl.BlockSpec((1, 1, L), lambda b: (b, 0, 0)),
                      pl.BlockSpec((L, D), lambda b: (b, 0))],
            out_specs=pl.BlockSpec((1, n_user, D), lambda b: (b, 0, 0)),
        ),
        compiler_params=pltpu.CompilerParams(
            dimension_semantics=("parallel",),
            vmem_limit_bytes=64 * 1024 * 1024),
    )(uidx.reshape(batch, 1, L), link_emb)

    return [out3[i] for i in range(batch)]


# speedup vs baseline: 2.5815x; 2.5815x over previous
import jax
import jax.numpy as jnp
from jax.experimental import pallas as pl
from jax.experimental.pallas import tpu as pltpu

_CORES = 2


def _mono_kernel(rawl_ref, sloc_ref, xany_ref, rawu_ref, su_b_ref, out_ref,
                 xtab_ref, lemb_ref, sem_ref, *, n_user, n_b, L, D, D_pad):
    c = pl.program_id(0)

    cp = pltpu.make_async_copy(xany_ref, xtab_ref, sem_ref)
    cp.start()
    cp.wait()

    lemb_ref[:, D:] = jnp.zeros((L, D_pad - D), jnp.float32)
    lemb_ref[:, D:D + 1] = jnp.ones((L, 1), jnp.float32)

    iota8 = jax.lax.broadcasted_iota(jnp.int32, (8, D), 0)
    iota_u = jax.lax.broadcasted_iota(jnp.int32, (n_user, L), 0)

    for bi in range(n_b):
        off = (c * n_b + bi) * L

        def _group(k, carry):
            acc = jnp.zeros((8, D), jnp.float32)
            for j in range(8):
                key = rawl_ref[off + 8 * k + j]
                li = sloc_ref[key]
                base = pl.multiple_of((li >> 3) << 3, 8)
                chunk = xtab_ref[pl.ds(base, 8), :]
                rolled = pltpu.roll(chunk, j - (li & 7), axis=0)
                acc = acc + jnp.where(iota8 == j, rolled, 0.0)
            lemb_ref[pl.ds(pl.multiple_of(8 * k, 8), 8), 0:D] = acc
            return carry

        jax.lax.fori_loop(0, L // 8, _group, 0)

        raw_u = rawu_ref[bi]
        rmask = iota_u == raw_u
        uv = jnp.sum(jnp.where(rmask, su_b_ref[...], 0), axis=0,
                     keepdims=True)

        oh = (iota_u == uv).astype(jnp.float32)
        sums = jnp.dot(oh, lemb_ref[...],
                       preferred_element_type=jnp.float32)
        counts = sums[:, D:D + 1]
        has = counts > 0.0
        avg = sums / jnp.maximum(counts, 1.0)
        n_edge = jnp.maximum(jnp.sum(has.astype(jnp.float32)), 1.0)
        mean_b = jnp.sum(avg, axis=0, keepdims=True) / n_edge
        res = jnp.where(has, avg, mean_b)
        out_ref[pl.ds(bi * n_user, n_user), :] = res[:, 0:D]


def kernel(x_location, x_mobility_batch, x_text_batch, sorted_user, sorted_location):
    x_m_t = jnp.concatenate([x_mobility_batch, x_text_batch], axis=2)
    links0 = x_m_t[:, 0]
    batch, L, _ = links0.shape
    n_loc, D = x_location.shape
    n_user = sorted_user.shape[0]
    n_b = batch // _CORES
    D_pad = 128 * pl.cdiv(D + 1, 128)

    rawu = links0[..., 0].astype(jnp.int32).reshape(batch, 1, L)
    rawl = links0[..., 1].astype(jnp.int32).reshape(batch * L)
    su_b = jnp.broadcast_to(sorted_user.astype(jnp.int32)[:, None], (n_user, L))

    import functools
    body = functools.partial(_mono_kernel, n_user=n_user, n_b=n_b, L=L, D=D,
                             D_pad=D_pad)
    out2 = pl.pallas_call(
        body,
        out_shape=jax.ShapeDtypeStruct((batch * n_user, D), jnp.float32),
        grid_spec=pltpu.PrefetchScalarGridSpec(
            num_scalar_prefetch=2,
            grid=(_CORES,),
            in_specs=[
                pl.BlockSpec(memory_space=pl.ANY),
                pl.BlockSpec((n_b, 1, L), lambda c, rl, sl: (c, 0, 0)),
                pl.BlockSpec((n_user, L), lambda c, rl, sl: (0, 0)),
            ],
            out_specs=pl.BlockSpec((n_b * n_user, D), lambda c, rl, sl: (c, 0)),
            scratch_shapes=[
                pltpu.VMEM((n_loc, D), jnp.float32),
                pltpu.VMEM((L, D_pad), jnp.float32),
                pltpu.SemaphoreType.DMA,
            ],
        ),
        compiler_params=pltpu.CompilerParams(
            dimension_semantics=("parallel",),
            vmem_limit_bytes=48 * 1024 * 1024),
    )(rawl, sorted_location.astype(jnp.int32), x_location, rawu, su_b)

    out3 = out2.reshape(batch, n_user, D)
    return [out3[i] for i in range(batch)]


# speedup vs baseline: 3.1749x; 1.2298x over previous
import functools

import jax
import jax.numpy as jnp
from jax.experimental import pallas as pl
from jax.experimental.pallas import tpu as pltpu

_CORES = 2


def _mono_kernel(rawl_ref, sloc_ref, xany_ref, rawu_ref, su_b_ref, out_ref,
                 xtab_ref, lemb_ref, sem_ref, *, n_user, n_b, L, D, D_pad):
    c = pl.program_id(0)

    cp = pltpu.make_async_copy(xany_ref, xtab_ref, sem_ref)
    cp.start()

    amask = su_b_ref[...] == jax.lax.broadcasted_iota(jnp.int32, (n_user, n_user), 1)
    iota_r = jax.lax.broadcasted_iota(jnp.int32, (n_user, n_user), 0)
    isu_row = jnp.sum(jnp.where(amask, iota_r, 0), axis=0, keepdims=True)

    lemb_ref[:, D:] = jnp.zeros((L, D_pad - D), jnp.float32)
    lemb_ref[:, D:D + 1] = jnp.ones((L, 1), jnp.float32)

    iota8 = jax.lax.broadcasted_iota(jnp.int32, (8, D), 0)
    ones_col = jnp.ones((n_user, 128), jnp.float32)

    cp.wait()

    for bi in range(n_b):
        off = (c * n_b + bi) * L

        def _group(k, carry):
            acc = jnp.zeros((8, D), jnp.float32)
            for j in range(8):
                key = rawl_ref[off + 8 * k + j]
                li = sloc_ref[key]
                base = pl.multiple_of((li >> 3) << 3, 8)
                chunk = xtab_ref[pl.ds(base, 8), :]
                rolled = pltpu.roll(chunk, j - (li & 7), axis=0)
                acc = acc + jnp.where(iota8 == j, rolled, 0.0)
            lemb_ref[pl.ds(pl.multiple_of(8 * k, 8), 8), 0:D] = acc
            return carry

        jax.lax.fori_loop(0, L // 8, _group, 0)

        oh_t = (rawu_ref[bi] == isu_row).astype(jnp.float32)
        sums_t = jax.lax.dot_general(
            lemb_ref[...], oh_t, (((0,), (0,)), ((), ())),
            preferred_element_type=jnp.float32)
        counts = sums_t[D:D + 1, :]
        has = counts > 0.0
        avg_t = sums_t * (1.0 / jnp.maximum(counts, 1.0))
        n_edge = jnp.maximum(jnp.sum(has.astype(jnp.float32)), 1.0)
        mean_c = jnp.dot(avg_t, ones_col,
                         preferred_element_type=jnp.float32)[:, 0:1] / n_edge
        res_t = jnp.where(has, avg_t, mean_c)
        out_ref[bi] = res_t[0:D, :]


def kernel(x_location, x_mobility_batch, x_text_batch, sorted_user, sorted_location):
    x_m_t = jnp.concatenate([x_mobility_batch, x_text_batch], axis=2)
    links0 = x_m_t[:, 0]
    batch, L, _ = links0.shape
    n_loc, D = x_location.shape
    n_user = sorted_user.shape[0]
    n_b = batch // _CORES
    D_pad = 128 * pl.cdiv(D + 1, 128)

    rawu = links0[..., 0].astype(jnp.int32).reshape(batch, L, 1)
    rawl = links0[..., 1].astype(jnp.int32).reshape(batch * L)
    su_b = jnp.broadcast_to(sorted_user.astype(jnp.int32)[:, None],
                            (n_user, n_user))

    body = functools.partial(_mono_kernel, n_user=n_user, n_b=n_b, L=L, D=D,
                             D_pad=D_pad)
    out4 = pl.pallas_call(
        body,
        out_shape=jax.ShapeDtypeStruct((batch, D, n_user), jnp.float32),
        grid_spec=pltpu.PrefetchScalarGridSpec(
            num_scalar_prefetch=2,
            grid=(_CORES,),
            in_specs=[
                pl.BlockSpec(memory_space=pl.ANY),
                pl.BlockSpec((batch // _CORES, L, 1), lambda c, rl, sl: (c, 0, 0)),
                pl.BlockSpec((n_user, n_user), lambda c, rl, sl: (0, 0)),
            ],
            out_specs=pl.BlockSpec((batch // _CORES, D, n_user),
                                   lambda c, rl, sl: (c, 0, 0)),
            scratch_shapes=[
                pltpu.VMEM((n_loc, D), jnp.float32),
                pltpu.VMEM((L, D_pad), jnp.float32),
                pltpu.SemaphoreType.DMA,
            ],
        ),
        compiler_params=pltpu.CompilerParams(
            dimension_semantics=("parallel",),
            vmem_limit_bytes=48 * 1024 * 1024),
    )(rawl, sorted_location.astype(jnp.int32), x_location, rawu, su_b)

    return [out4[i].T for i in range(batch)]
